# Initial kernel scaffold; baseline (speedup 1.0000x reference)
import jax
import jax.numpy as jnp
from jax import lax
from jax.experimental import pallas as pl
from jax.experimental.pallas import tpu as pltpu
from jax.experimental.pallas import tpu_sc as plsc

_N = 10000
_E = 320000
_D = 128
_R = 200
_SLOPE = (1.0 / 8.0 + 1.0 / 3.0) / 2.0
_NC = 2
_NS = 16
_NW = _NC * _NS
_EPT = _E // _NW
_CHUNK = 80
_NCHUNK = _EPT // _CHUNK
_RPT = _N // _NS
_NT = _N + _R


def _mm_block(x_ref, w1_ref, w2_ref, o1_ref, o2_ref):
    x = x_ref[...]
    o1_ref[...] = jnp.dot(x, w1_ref[...], preferred_element_type=jnp.float32)
    o2_ref[...] = jnp.dot(x, w2_ref[...], preferred_element_type=jnp.float32)


def _mm2(x, w1, w2):
    rows = x.shape[0]
    bm = 1024
    grid = pl.cdiv(rows, bm)
    return pl.pallas_call(
        _mm_block,
        grid=(grid,),
        in_specs=[
            pl.BlockSpec((bm, _D), lambda i: (i, 0)),
            pl.BlockSpec((_D, w1.shape[1]), lambda i: (0, 0)),
            pl.BlockSpec((_D, w2.shape[1]), lambda i: (0, 0)),
        ],
        out_specs=[
            pl.BlockSpec((bm, w1.shape[1]), lambda i: (i, 0)),
            pl.BlockSpec((bm, w2.shape[1]), lambda i: (i, 0)),
        ],
        out_shape=[
            jax.ShapeDtypeStruct((rows, w1.shape[1]), jnp.float32),
            jax.ShapeDtypeStruct((rows, w2.shape[1]), jnp.float32),
        ],
    )(x, w1, w2)


def _apply_block(p_ref, deg_ref, norm_ref, si_ref, o_ref):
    agg = p_ref[0] + p_ref[1]
    d = deg_ref[...]
    deg = d[0, :, :1] + d[1, :, :1]
    iso = deg == 0.0
    si = si_ref[...]
    sm = jnp.where(iso, si[:, _D:], si[:, :_D])
    pre = agg * norm_ref[...] + sm
    o_ref[...] = jnp.where(pre >= 0.0, pre, jnp.float32(_SLOPE) * pre)


def _apply(parts, deg, norm, si):
    bm = 500
    grid = _N // bm
    return pl.pallas_call(
        _apply_block,
        grid=(grid,),
        in_specs=[
            pl.BlockSpec((2, bm, _D), lambda i: (0, i, 0)),
            pl.BlockSpec((2, bm, 16), lambda i: (0, i, 0)),
            pl.BlockSpec((bm, 1), lambda i: (i, 0)),
            pl.BlockSpec((bm, 2 * _D), lambda i: (i, 0)),
        ],
        out_specs=pl.BlockSpec((bm, _D), lambda i: (i, 0)),
        out_shape=jax.ShapeDtypeStruct((_N, _D), jnp.float32),
    )(parts, deg, norm, si)


def _make_sc(with_deg):
    outs = [jax.ShapeDtypeStruct((_NC, _N, _D), jnp.float32)]
    scratch = [
        pltpu.VMEM((_NCHUNK, _CHUNK), jnp.int32),
        pltpu.VMEM((_NCHUNK, _CHUNK), jnp.int32),
        pltpu.VMEM((_NCHUNK, _CHUNK), jnp.int32),
        pltpu.VMEM((_CHUNK, _D), jnp.float32),
        pltpu.VMEM((_CHUNK, _D), jnp.float32),
        pltpu.VMEM_SHARED((_N, _D), jnp.float32),
    ]
    if with_deg:
        outs.append(jax.ShapeDtypeStruct((_NC, _N, 16), jnp.float32))
        scratch += [
            pltpu.VMEM((_CHUNK, 16), jnp.float32),
            pltpu.VMEM_SHARED((_N, 16), jnp.float32),
        ]
    scratch += [pltpu.SemaphoreType.DMA, pltpu.SemaphoreType.DMA]
    mesh = plsc.VectorSubcoreMesh(core_axis_name="c", subcore_axis_name="s")

    def body(*refs):
        if with_deg:
            (t_hbm, src_hbm, rel_hbm, dst_hbm, z128_hbm, z16_hbm, ones_hbm,
             agg_hbm, deg_hbm,
             src_v, rel_v, dst_v, buf_a, buf_b, acc_sh, ones_v, deg_sh,
             sem_a, sem_b) = refs
        else:
            (t_hbm, src_hbm, rel_hbm, dst_hbm, z128_hbm,
             agg_hbm,
             src_v, rel_v, dst_v, buf_a, buf_b, acc_sh,
             sem_a, sem_b) = refs
        c = lax.axis_index("c")
        s = lax.axis_index("s")
        wid = s * _NC + c
        rbase = s * _RPT

        pltpu.sync_copy(src_hbm.at[wid], src_v)
        pltpu.sync_copy(rel_hbm.at[wid], rel_v)
        pltpu.sync_copy(dst_hbm.at[wid], dst_v)
        pltpu.sync_copy(z128_hbm.at[pl.ds(rbase, _RPT)],
                        acc_sh.at[pl.ds(rbase, _RPT)])
        if with_deg:
            pltpu.sync_copy(z16_hbm.at[pl.ds(rbase, _RPT)],
                            deg_sh.at[pl.ds(rbase, _RPT)])
            pltpu.sync_copy(ones_hbm, ones_v)
        plsc.subcore_barrier()

        @pl.loop(0, _NCHUNK)
        def _(i):
            ga = pltpu.async_copy(t_hbm.at[src_v.at[i]], buf_a, sem_a)
            gb = pltpu.async_copy(t_hbm.at[rel_v.at[i]], buf_b, sem_b)
            ga.wait()
            gb.wait()
            pltpu.sync_copy(buf_a, acc_sh.at[dst_v.at[i]], add=True)
            pltpu.sync_copy(buf_b, acc_sh.at[dst_v.at[i]], add=True)
            if with_deg:
                pltpu.sync_copy(ones_v, deg_sh.at[dst_v.at[i]], add=True)

        plsc.subcore_barrier()
        pltpu.sync_copy(acc_sh.at[pl.ds(rbase, _RPT)],
                        agg_hbm.at[c, pl.ds(rbase, _RPT)])
        if with_deg:
            pltpu.sync_copy(deg_sh.at[pl.ds(rbase, _RPT)],
                            deg_hbm.at[c, pl.ds(rbase, _RPT)])

    return pl.kernel(body, out_type=outs, mesh=mesh, scratch_types=scratch)


_sc_deg = _make_sc(True)
_sc_nodeg = _make_sc(False)


@jax.jit
def kernel(ent_embed, rel_embed, norm, edge_index, rel_id,
           W_r0, W_sl0, W_el0, W_r1, W_sl1, W_el1):
    src = edge_index[0].reshape(_NW, _NCHUNK, _CHUNK)
    dst = edge_index[1].reshape(_NW, _NCHUNK, _CHUNK)
    rel = (rel_id + _N).astype(jnp.int32).reshape(_NW, _NCHUNK, _CHUNK)
    z128 = jnp.zeros((_N, _D), jnp.float32)
    z16 = jnp.zeros((_N, 16), jnp.float32)
    ones16 = jnp.ones((_CHUNK, 16), jnp.float32)
    w1_0 = jnp.concatenate([W_sl0.T, W_el0.T], axis=1)
    w1_1 = jnp.concatenate([W_sl1.T, W_el1.T], axis=1)

    x0 = jnp.concatenate([ent_embed, rel_embed], axis=0)
    si0, t0 = _mm2(x0, w1_0, W_r0.T)
    agg0, deg = _sc_deg(t0, src, rel, dst, z128, z16, ones16)
    h1 = _apply(agg0, deg, norm, si0)

    x1 = jnp.concatenate([h1, rel_embed], axis=0)
    si1, t1 = _mm2(x1, w1_1, W_r1.T)
    (agg1,) = _sc_nodeg(t1, src, rel, dst, z128)
    h2 = _apply(agg1, deg, norm, si1)
    return h2


# speedup vs baseline: 4.3523x; 4.3523x over previous
import dataclasses
import jax
import jax.numpy as jnp
from jax import lax
from jax.experimental import pallas as pl
from jax.experimental.pallas import tpu as pltpu
from jax.experimental.pallas import tpu_sc as plsc

_N = 10000
_E = 320000
_D = 128
_R = 200
_SLOPE = (1.0 / 8.0 + 1.0 / 3.0) / 2.0
_NC = 2
_NS = 16
_NW = _NC * _NS
_EPT = _E // _NW
_CHUNK = 80
_NCHUNK = _EPT // _CHUNK
_NP = 10240
_RPT = _NP // _NS
_NT = _N + _R

def _sc_compiler_params():
    cp = pltpu.CompilerParams()
    if "needs_layout_passes" in pltpu.CompilerParams.__dataclass_fields__:
        cp = dataclasses.replace(cp, needs_layout_passes=False)
    return cp


def _mm_block(x_ref, w1_ref, w2_ref, o1_ref, o2_ref):
    x = x_ref[...]
    o1_ref[...] = jnp.dot(x, w1_ref[...], preferred_element_type=jnp.float32)
    o2_ref[...] = jnp.dot(x, w2_ref[...], preferred_element_type=jnp.float32)


def _mm2(x, w1, w2):
    rows = x.shape[0]
    bm = 1024
    grid = pl.cdiv(rows, bm)
    return pl.pallas_call(
        _mm_block,
        grid=(grid,),
        in_specs=[
            pl.BlockSpec((bm, _D), lambda i: (i, 0)),
            pl.BlockSpec((_D, w1.shape[1]), lambda i: (0, 0)),
            pl.BlockSpec((_D, w2.shape[1]), lambda i: (0, 0)),
        ],
        out_specs=[
            pl.BlockSpec((bm, w1.shape[1]), lambda i: (i, 0)),
            pl.BlockSpec((bm, w2.shape[1]), lambda i: (i, 0)),
        ],
        out_shape=[
            jax.ShapeDtypeStruct((rows, w1.shape[1]), jnp.float32),
            jax.ShapeDtypeStruct((rows, w2.shape[1]), jnp.float32),
        ],
    )(x, w1, w2)


def _apply_block(p_ref, deg_ref, norm_ref, si_ref, o_ref):
    agg = p_ref[0] + p_ref[1]
    deg = jnp.sum(deg_ref[...], axis=0)[:, None]
    iso = deg == 0.0
    si = si_ref[...]
    sm = jnp.where(iso, si[:, _D:], si[:, :_D])
    pre = agg * norm_ref[...] + sm
    o_ref[...] = jnp.where(pre >= 0.0, pre, jnp.float32(_SLOPE) * pre)


def _apply(parts, deg, norm, si):
    bm = 1024
    grid = pl.cdiv(_N, bm)
    return pl.pallas_call(
        _apply_block,
        grid=(grid,),
        in_specs=[
            pl.BlockSpec((2, bm, _D), lambda i: (0, i, 0)),
            pl.BlockSpec((_NW, bm), lambda i: (0, i)),
            pl.BlockSpec((bm, 1), lambda i: (i, 0)),
            pl.BlockSpec((bm, 2 * _D), lambda i: (i, 0)),
        ],
        out_specs=pl.BlockSpec((bm, _D), lambda i: (i, 0)),
        out_shape=jax.ShapeDtypeStruct((_N, _D), jnp.float32),
    )(parts, deg, norm, si)


def _make_sc(with_deg):
    outs = [jax.ShapeDtypeStruct((_NC * _NP, _D), jnp.float32)]
    scratch = [
        pltpu.VMEM((_CHUNK,), jnp.int32),
        pltpu.VMEM((_CHUNK,), jnp.int32),
        pltpu.VMEM((_CHUNK,), jnp.int32),
        pltpu.VMEM((_CHUNK, _D), jnp.float32),
        pltpu.VMEM((_CHUNK, _D), jnp.float32),
        pltpu.VMEM_SHARED((_NP, _D), jnp.float32),
    ]
    if with_deg:
        outs.append(jax.ShapeDtypeStruct((_NW * _N,), jnp.float32))
        scratch.append(pltpu.VMEM((_N,), jnp.float32))
    scratch += [pltpu.SemaphoreType.DMA, pltpu.SemaphoreType.DMA]
    mesh = plsc.VectorSubcoreMesh(core_axis_name="c", subcore_axis_name="s")

    def body(*refs):
        if with_deg:
            (t_hbm, src_hbm, rel_hbm, dst_hbm, z128_hbm, z1_hbm,
             agg_hbm, deg_hbm,
             src_v, rel_v, dst_v, buf_a, buf_b, acc_sh, deg_t,
             sem_a, sem_b) = refs
        else:
            (t_hbm, src_hbm, rel_hbm, dst_hbm, z128_hbm,
             agg_hbm,
             src_v, rel_v, dst_v, buf_a, buf_b, acc_sh,
             sem_a, sem_b) = refs
        c = lax.axis_index("c")
        s = lax.axis_index("s")
        wid = s * _NC + c
        rbase = s * _RPT

        pltpu.sync_copy(z128_hbm.at[pl.ds(rbase, _RPT)],
                        acc_sh.at[pl.ds(rbase, _RPT)])
        if with_deg:
            pltpu.sync_copy(z1_hbm, deg_t)
        plsc.subcore_barrier()

        @pl.loop(0, _NCHUNK)
        def _(i):
            ebase = wid * _EPT + i * _CHUNK
            pltpu.sync_copy(src_hbm.at[pl.ds(ebase, _CHUNK)], src_v)
            pltpu.sync_copy(rel_hbm.at[pl.ds(ebase, _CHUNK)], rel_v)
            pltpu.sync_copy(dst_hbm.at[pl.ds(ebase, _CHUNK)], dst_v)
            ga = pltpu.async_copy(t_hbm.at[src_v], buf_a, sem_a)
            gb = pltpu.async_copy(t_hbm.at[rel_v], buf_b, sem_b)
            ga.wait()
            gb.wait()
            pltpu.sync_copy(buf_a, acc_sh.at[dst_v], add=True)
            pltpu.sync_copy(buf_b, acc_sh.at[dst_v], add=True)
            if with_deg:
                for j in range(_CHUNK // 16):
                    idx16 = dst_v[pl.ds(j * 16, 16)]
                    plsc.addupdate_scatter(deg_t, [idx16],
                                           jnp.full((16,), 1.0, jnp.float32))

        plsc.subcore_barrier()
        pltpu.sync_copy(acc_sh.at[pl.ds(rbase, _RPT)],
                        agg_hbm.at[pl.ds(c * _NP + rbase, _RPT)])
        if with_deg:
            pltpu.sync_copy(deg_t, deg_hbm.at[pl.ds(wid * _N, _N)])

    kwargs = {}
    if with_deg:
        kwargs["compiler_params"] = _sc_compiler_params()
    return pl.kernel(body, out_type=outs, mesh=mesh, scratch_types=scratch,
                     **kwargs)


_SC_CACHE = {}


def _sc(with_deg):
    if with_deg not in _SC_CACHE:
        _SC_CACHE[with_deg] = _make_sc(with_deg)
    return _SC_CACHE[with_deg]


@jax.jit
def kernel(ent_embed, rel_embed, norm, edge_index, rel_id,
           W_r0, W_sl0, W_el0, W_r1, W_sl1, W_el1):
    src = edge_index[0]
    dst = edge_index[1]
    rel = (rel_id + _N).astype(jnp.int32)
    z128 = jnp.zeros((_NP, _D), jnp.float32)
    z1 = jnp.zeros((_N,), jnp.float32)
    w1_0 = jnp.concatenate([W_sl0.T, W_el0.T], axis=1)
    w1_1 = jnp.concatenate([W_sl1.T, W_el1.T], axis=1)

    x0 = jnp.concatenate([ent_embed, rel_embed], axis=0)
    si0, t0 = _mm2(x0, w1_0, W_r0.T)
    agg0, deg = _sc(True)(t0, src, rel, dst, z128, z1)
    agg0 = agg0.reshape(_NC, _NP, _D)
    deg = deg.reshape(_NW, _N)
    h1 = _apply(agg0, deg, norm, si0)

    x1 = jnp.concatenate([h1, rel_embed], axis=0)
    si1, t1 = _mm2(x1, w1_1, W_r1.T)
    (agg1,) = _sc(False)(t1, src, rel, dst, z128)
    agg1 = agg1.reshape(_NC, _NP, _D)
    h2 = _apply(agg1, deg, norm, si1)
    return h2


# speedup vs baseline: 6.2590x; 1.4381x over previous
import dataclasses
import jax
import jax.numpy as jnp
from jax import lax
from jax.experimental import pallas as pl
from jax.experimental.pallas import tpu as pltpu
from jax.experimental.pallas import tpu_sc as plsc

_N = 10000
_E = 320000
_D = 128
_R = 200
_SLOPE = (1.0 / 8.0 + 1.0 / 3.0) / 2.0
_NC = 2
_NS = 16
_NW = _NC * _NS
_EPT = _E // _NW
_CHUNK = 80
_NCHUNK = _EPT // _CHUNK
_RPT = 624
_CHUNK_DEG = 400
_NT = _N + _R

def _sc_compiler_params():
    cp = pltpu.CompilerParams()
    if "needs_layout_passes" in pltpu.CompilerParams.__dataclass_fields__:
        cp = dataclasses.replace(cp, needs_layout_passes=False)
    return cp


def _mm_block(x_ref, w1_ref, w2_ref, o1_ref, o2_ref):
    x = x_ref[...]
    o1_ref[...] = jnp.dot(x, w1_ref[...], preferred_element_type=jnp.float32)
    o2_ref[...] = jnp.dot(x, w2_ref[...], preferred_element_type=jnp.float32)


def _mm2(x, w1, w2):
    rows = x.shape[0]
    bm = 1024
    grid = pl.cdiv(rows, bm)
    return pl.pallas_call(
        _mm_block,
        grid=(grid,),
        in_specs=[
            pl.BlockSpec((bm, _D), lambda i: (i, 0)),
            pl.BlockSpec((_D, w1.shape[1]), lambda i: (0, 0)),
            pl.BlockSpec((_D, w2.shape[1]), lambda i: (0, 0)),
        ],
        out_specs=[
            pl.BlockSpec((bm, w1.shape[1]), lambda i: (i, 0)),
            pl.BlockSpec((bm, w2.shape[1]), lambda i: (i, 0)),
        ],
        out_shape=[
            jax.ShapeDtypeStruct((rows, w1.shape[1]), jnp.float32),
            jax.ShapeDtypeStruct((rows, w2.shape[1]), jnp.float32),
        ],
    )(x, w1, w2)


def _apply_block(p_ref, deg_ref, norm_ref, si_ref, o_ref):
    agg = p_ref[0] + p_ref[1]
    deg = jnp.sum(deg_ref[...], axis=0)[:, None]
    iso = deg == 0.0
    si = si_ref[...]
    sm = jnp.where(iso, si[:, _D:], si[:, :_D])
    pre = agg * norm_ref[...] + sm
    o_ref[...] = jnp.where(pre >= 0.0, pre, jnp.float32(_SLOPE) * pre)


def _apply(parts, deg, norm, si):
    bm = 1024
    grid = pl.cdiv(_N, bm)
    return pl.pallas_call(
        _apply_block,
        grid=(grid,),
        in_specs=[
            pl.BlockSpec((2, bm, _D), lambda i: (0, i, 0)),
            pl.BlockSpec((_NW, bm), lambda i: (0, i)),
            pl.BlockSpec((bm, 1), lambda i: (i, 0)),
            pl.BlockSpec((bm, 2 * _D), lambda i: (i, 0)),
        ],
        out_specs=pl.BlockSpec((bm, _D), lambda i: (i, 0)),
        out_shape=jax.ShapeDtypeStruct((_N, _D), jnp.float32),
    )(parts, deg, norm, si)


def _make_deg():
    mesh = plsc.VectorSubcoreMesh(core_axis_name="c", subcore_axis_name="s")
    outs = [jax.ShapeDtypeStruct((_NW * _N,), jnp.float32)]
    scratch = [
        pltpu.VMEM((_CHUNK_DEG,), jnp.int32),
        pltpu.VMEM((_N,), jnp.float32),
    ]

    def body(dst_hbm, z1_hbm, deg_hbm, dst_v, deg_t):
        c = lax.axis_index("c")
        s = lax.axis_index("s")
        wid = s * _NC + c
        pltpu.sync_copy(z1_hbm, deg_t)

        @pl.loop(0, _EPT // _CHUNK_DEG)
        def _(i):
            ebase = wid * _EPT + i * _CHUNK_DEG
            pltpu.sync_copy(dst_hbm.at[pl.ds(ebase, _CHUNK_DEG)], dst_v)
            for j in range(_CHUNK_DEG // 16):
                idx16 = dst_v[pl.ds(j * 16, 16)]
                plsc.addupdate_scatter(deg_t, [idx16],
                                       jnp.full((16,), 1.0, jnp.float32))

        pltpu.sync_copy(deg_t, deg_hbm.at[pl.ds(wid * _N, _N)])

    return pl.kernel(body, out_type=outs, mesh=mesh, scratch_types=scratch,
                     compiler_params=_sc_compiler_params())


def _make_sc():
    outs = [jax.ShapeDtypeStruct((_NC * _N, _D), jnp.float32)]
    scratch = []
    for _slot in range(2):
        scratch += [
            pltpu.VMEM((_CHUNK,), jnp.int32),
            pltpu.VMEM((_CHUNK,), jnp.int32),
            pltpu.VMEM((_CHUNK,), jnp.int32),
            pltpu.VMEM((_CHUNK, _D), jnp.float32),
            pltpu.VMEM((_CHUNK, _D), jnp.float32),
            pltpu.SemaphoreType.DMA,
            pltpu.SemaphoreType.DMA,
        ]
    scratch.append(pltpu.VMEM_SHARED((_N, _D), jnp.float32))
    mesh = plsc.VectorSubcoreMesh(core_axis_name="c", subcore_axis_name="s")

    def body(t_hbm, src_hbm, rel_hbm, dst_hbm, z128_hbm, agg_hbm,
             src0, rel0, dst0, bufa0, bufb0, sa0, sb0,
             src1, rel1, dst1, bufa1, bufb1, sa1, sb1,
             acc_sh):
        c = lax.axis_index("c")
        s = lax.axis_index("s")
        wid = s * _NC + c
        rbase = s * _RPT
        slots = ((src0, rel0, dst0, bufa0, bufb0, sa0, sb0),
                 (src1, rel1, dst1, bufa1, bufb1, sa1, sb1))

        pltpu.sync_copy(z128_hbm.at[pl.ds(rbase, _RPT)],
                        acc_sh.at[pl.ds(rbase, _RPT)])

        @pl.when(s == _NS - 1)
        def _():
            pltpu.sync_copy(z128_hbm.at[pl.ds(_NS * _RPT, _N - _NS * _RPT)],
                            acc_sh.at[pl.ds(_NS * _RPT, _N - _NS * _RPT)])

        plsc.subcore_barrier()

        def load_idx(k, i):
            src_v, rel_v, dst_v = slots[k][0], slots[k][1], slots[k][2]
            ebase = wid * _EPT + i * _CHUNK
            pltpu.sync_copy(src_hbm.at[pl.ds(ebase, _CHUNK)], src_v)
            pltpu.sync_copy(rel_hbm.at[pl.ds(ebase, _CHUNK)], rel_v)
            pltpu.sync_copy(dst_hbm.at[pl.ds(ebase, _CHUNK)], dst_v)

        def start_gathers(k):
            src_v, rel_v, _, buf_a, buf_b, sem_a, sem_b = slots[k]
            pltpu.async_copy(t_hbm.at[src_v], buf_a, sem_a)
            pltpu.async_copy(t_hbm.at[rel_v], buf_b, sem_b)

        def wait_gathers(k):
            src_v, rel_v, _, buf_a, buf_b, sem_a, sem_b = slots[k]
            pltpu.make_async_copy(t_hbm.at[src_v], buf_a, sem_a).wait()
            pltpu.make_async_copy(t_hbm.at[rel_v], buf_b, sem_b).wait()

        def scatter(k):
            dst_v, buf_a, buf_b = slots[k][2], slots[k][3], slots[k][4]
            pltpu.sync_copy(buf_a, acc_sh.at[dst_v], add=True)
            pltpu.sync_copy(buf_b, acc_sh.at[dst_v], add=True)

        load_idx(0, 0)
        start_gathers(0)

        @pl.loop(0, _NCHUNK + 1, step=2)
        def _(i):
            @pl.when(i + 1 < _NCHUNK)
            def _():
                load_idx(1, i + 1)
                start_gathers(1)

            wait_gathers(0)
            scatter(0)

            @pl.when(i + 2 < _NCHUNK)
            def _():
                load_idx(0, i + 2)
                start_gathers(0)

            @pl.when(i + 1 < _NCHUNK)
            def _():
                wait_gathers(1)
                scatter(1)

        plsc.subcore_barrier()
        pltpu.sync_copy(acc_sh.at[pl.ds(rbase, _RPT)],
                        agg_hbm.at[pl.ds(c * _N + rbase, _RPT)])

        @pl.when(s == _NS - 1)
        def _():
            pltpu.sync_copy(acc_sh.at[pl.ds(_NS * _RPT, _N - _NS * _RPT)],
                            agg_hbm.at[pl.ds(c * _N + _NS * _RPT,
                                             _N - _NS * _RPT)])

    return pl.kernel(body, out_type=outs, mesh=mesh, scratch_types=scratch)


_SC_CACHE = {}


def _sc(kind):
    if kind not in _SC_CACHE:
        _SC_CACHE[kind] = _make_deg() if kind == "deg" else _make_sc()
    return _SC_CACHE[kind]


@jax.jit
def kernel(ent_embed, rel_embed, norm, edge_index, rel_id,
           W_r0, W_sl0, W_el0, W_r1, W_sl1, W_el1):
    src = edge_index[0]
    dst = edge_index[1]
    rel = (rel_id + _N).astype(jnp.int32)
    z128 = jnp.zeros((_N, _D), jnp.float32)
    z1 = jnp.zeros((_N,), jnp.float32)
    w1_0 = jnp.concatenate([W_sl0.T, W_el0.T], axis=1)
    w1_1 = jnp.concatenate([W_sl1.T, W_el1.T], axis=1)

    (deg,) = _sc("deg")(dst, z1)
    deg = deg.reshape(_NW, _N)

    x0 = jnp.concatenate([ent_embed, rel_embed], axis=0)
    si0, t0 = _mm2(x0, w1_0, W_r0.T)
    (agg0,) = _sc("main")(t0, src, rel, dst, z128)
    agg0 = agg0.reshape(_NC, _N, _D)
    h1 = _apply(agg0, deg, norm, si0)

    x1 = jnp.concatenate([h1, rel_embed], axis=0)
    si1, t1 = _mm2(x1, w1_1, W_r1.T)
    (agg1,) = _sc("main")(t1, src, rel, dst, z128)
    agg1 = agg1.reshape(_NC, _N, _D)
    h2 = _apply(agg1, deg, norm, si1)
    return h2


# speedup vs baseline: 7.0332x; 1.1237x over previous
import dataclasses
import jax
import jax.numpy as jnp
from jax import lax
from jax.experimental import pallas as pl
from jax.experimental.pallas import tpu as pltpu
from jax.experimental.pallas import tpu_sc as plsc

_N = 10000
_E = 320000
_D = 128
_R = 200
_SLOPE = (1.0 / 8.0 + 1.0 / 3.0) / 2.0
_NC = 2
_NS = 16
_NW = _NC * _NS
_EPT = _E // _NW
_CHUNK = 80
_NCHUNK = _EPT // _CHUNK
_RPT = 624
_CHUNK_DEG = 400
_NT = _N + _R

def _sc_compiler_params():
    cp = pltpu.CompilerParams()
    if "needs_layout_passes" in pltpu.CompilerParams.__dataclass_fields__:
        cp = dataclasses.replace(cp, needs_layout_passes=False)
    return cp


def _mm_block(x_ref, w1_ref, w2_ref, o1_ref, o2_ref):
    x = x_ref[...]
    o1_ref[...] = jnp.dot(x, w1_ref[...], preferred_element_type=jnp.float32)
    o2_ref[...] = jnp.dot(x, w2_ref[...], preferred_element_type=jnp.float32)


def _mm2(x, w1, w2):
    rows = x.shape[0]
    bm = 1024
    grid = pl.cdiv(rows, bm)
    return pl.pallas_call(
        _mm_block,
        grid=(grid,),
        in_specs=[
            pl.BlockSpec((bm, _D), lambda i: (i, 0)),
            pl.BlockSpec((_D, w1.shape[1]), lambda i: (0, 0)),
            pl.BlockSpec((_D, w2.shape[1]), lambda i: (0, 0)),
        ],
        out_specs=[
            pl.BlockSpec((bm, w1.shape[1]), lambda i: (i, 0)),
            pl.BlockSpec((bm, w2.shape[1]), lambda i: (i, 0)),
        ],
        out_shape=[
            jax.ShapeDtypeStruct((rows, w1.shape[1]), jnp.float32),
            jax.ShapeDtypeStruct((rows, w2.shape[1]), jnp.float32),
        ],
    )(x, w1, w2)


def _apply_block(p_ref, deg_ref, norm_ref, si_ref, o_ref):
    agg = p_ref[0] + p_ref[1]
    deg = jnp.sum(deg_ref[...], axis=0)[:, None]
    iso = deg == 0.0
    si = si_ref[...]
    sm = jnp.where(iso, si[:, _D:], si[:, :_D])
    pre = agg * norm_ref[...] + sm
    o_ref[...] = jnp.where(pre >= 0.0, pre, jnp.float32(_SLOPE) * pre)


def _apply(parts, deg, norm, si):
    bm = 1024
    grid = pl.cdiv(_N, bm)
    return pl.pallas_call(
        _apply_block,
        grid=(grid,),
        in_specs=[
            pl.BlockSpec((2, bm, _D), lambda i: (0, i, 0)),
            pl.BlockSpec((_NW, bm), lambda i: (0, i)),
            pl.BlockSpec((bm, 1), lambda i: (i, 0)),
            pl.BlockSpec((bm, 2 * _D), lambda i: (i, 0)),
        ],
        out_specs=pl.BlockSpec((bm, _D), lambda i: (i, 0)),
        out_shape=jax.ShapeDtypeStruct((_N, _D), jnp.float32),
    )(parts, deg, norm, si)


def _make_deg():
    mesh = plsc.VectorSubcoreMesh(core_axis_name="c", subcore_axis_name="s")
    outs = [jax.ShapeDtypeStruct((_NW * _N,), jnp.float32)]
    scratch = [
        pltpu.VMEM((_CHUNK_DEG,), jnp.int32),
        pltpu.VMEM((_N,), jnp.float32),
    ]

    def body(dst_hbm, z1_hbm, deg_hbm, dst_v, deg_t):
        c = lax.axis_index("c")
        s = lax.axis_index("s")
        wid = s * _NC + c
        pltpu.sync_copy(z1_hbm, deg_t)

        @pl.loop(0, _EPT // _CHUNK_DEG)
        def _(i):
            ebase = wid * _EPT + i * _CHUNK_DEG
            pltpu.sync_copy(dst_hbm.at[pl.ds(ebase, _CHUNK_DEG)], dst_v)
            for j in range(_CHUNK_DEG // 16):
                idx16 = dst_v[pl.ds(j * 16, 16)]
                plsc.addupdate_scatter(deg_t, [idx16],
                                       jnp.full((16,), 1.0, jnp.float32))

        pltpu.sync_copy(deg_t, deg_hbm.at[pl.ds(wid * _N, _N)])

    return pl.kernel(body, out_type=outs, mesh=mesh, scratch_types=scratch,
                     compiler_params=_sc_compiler_params())


def _make_sc():
    outs = [jax.ShapeDtypeStruct((_NC * _N, _D), jnp.float32)]
    scratch = []
    for _slot in range(2):
        scratch += [
            pltpu.VMEM((2, _CHUNK), jnp.int32),
            pltpu.VMEM((_CHUNK,), jnp.int32),
            pltpu.VMEM((_CHUNK, _D), jnp.float32),
            pltpu.VMEM((_CHUNK, _D), jnp.float32),
            pltpu.SemaphoreType.DMA,
            pltpu.SemaphoreType.DMA,
            pltpu.SemaphoreType.DMA,
        ]
    scratch.append(pltpu.VMEM_SHARED((_N, _D), jnp.float32))
    mesh = plsc.VectorSubcoreMesh(core_axis_name="c", subcore_axis_name="s")

    def body(t_hbm, srel_hbm, dst_hbm, z128_hbm, agg_hbm,
             isr0, dst0, bufa0, bufb0, sa0, sb0, ss0,
             isr1, dst1, bufa1, bufb1, sa1, sb1, ss1,
             acc_sh):
        c = lax.axis_index("c")
        s = lax.axis_index("s")
        wid = s * _NC + c
        rbase = s * _RPT
        slots = ((isr0, dst0, bufa0, bufb0, sa0, sb0, ss0),
                 (isr1, dst1, bufa1, bufb1, sa1, sb1, ss1))

        pltpu.sync_copy(z128_hbm.at[pl.ds(rbase, _RPT)],
                        acc_sh.at[pl.ds(rbase, _RPT)])

        @pl.when(s == _NS - 1)
        def _():
            pltpu.sync_copy(z128_hbm.at[pl.ds(_NS * _RPT, _N - _NS * _RPT)],
                            acc_sh.at[pl.ds(_NS * _RPT, _N - _NS * _RPT)])

        plsc.subcore_barrier()

        def load_idx(k, i):
            isr_v, dst_v = slots[k][0], slots[k][1]
            row = wid * _NCHUNK + i
            pltpu.sync_copy(srel_hbm.at[row], isr_v)
            pltpu.sync_copy(dst_hbm.at[pl.ds(wid * _EPT + i * _CHUNK, _CHUNK)],
                            dst_v)

        def start_gathers(k):
            isr_v, _, buf_a, buf_b, sem_a, sem_b, _ = slots[k]
            pltpu.async_copy(t_hbm.at[isr_v.at[0]], buf_a, sem_a)
            pltpu.async_copy(t_hbm.at[isr_v.at[1]], buf_b, sem_b)

        def wait_gathers(k):
            isr_v, _, buf_a, buf_b, sem_a, sem_b, _ = slots[k]
            pltpu.make_async_copy(t_hbm.at[isr_v.at[0]], buf_a, sem_a).wait()
            pltpu.make_async_copy(t_hbm.at[isr_v.at[1]], buf_b, sem_b).wait()

        def start_scatters(k):
            _, dst_v, buf_a, buf_b, _, _, sem_s = slots[k]
            pltpu.async_copy(buf_a, acc_sh.at[dst_v], sem_s, add=True)
            pltpu.async_copy(buf_b, acc_sh.at[dst_v], sem_s, add=True)

        def wait_scatters(k):
            _, dst_v, buf_a, buf_b, _, _, sem_s = slots[k]
            pltpu.make_async_copy(buf_a, acc_sh.at[dst_v], sem_s).wait()
            pltpu.make_async_copy(buf_b, acc_sh.at[dst_v], sem_s).wait()

        load_idx(0, 0)
        start_gathers(0)
        load_idx(1, 1)
        start_gathers(1)

        @pl.loop(0, _NCHUNK + 1, step=2)
        def _(i):
            wait_gathers(0)
            start_scatters(0)

            @pl.when(i + 1 < _NCHUNK)
            def _():
                wait_gathers(1)
                start_scatters(1)

            @pl.when(i + 2 < _NCHUNK)
            def _():
                wait_scatters(0)
                load_idx(0, i + 2)
                start_gathers(0)

            @pl.when(i + 3 < _NCHUNK)
            def _():
                wait_scatters(1)
                load_idx(1, i + 3)
                start_gathers(1)

        wait_scatters(1)
        wait_scatters(0)

        plsc.subcore_barrier()
        pltpu.sync_copy(acc_sh.at[pl.ds(rbase, _RPT)],
                        agg_hbm.at[pl.ds(c * _N + rbase, _RPT)])

        @pl.when(s == _NS - 1)
        def _():
            pltpu.sync_copy(acc_sh.at[pl.ds(_NS * _RPT, _N - _NS * _RPT)],
                            agg_hbm.at[pl.ds(c * _N + _NS * _RPT,
                                             _N - _NS * _RPT)])

    return pl.kernel(body, out_type=outs, mesh=mesh, scratch_types=scratch)


_SC_CACHE = {}


def _sc(kind):
    if kind not in _SC_CACHE:
        _SC_CACHE[kind] = _make_deg() if kind == "deg" else _make_sc()
    return _SC_CACHE[kind]


@jax.jit
def kernel(ent_embed, rel_embed, norm, edge_index, rel_id,
           W_r0, W_sl0, W_el0, W_r1, W_sl1, W_el1):
    dst = edge_index[1]
    srel = jnp.stack([edge_index[0].reshape(_NW * _NCHUNK, _CHUNK),
                      (rel_id + _N).astype(jnp.int32).reshape(
                          _NW * _NCHUNK, _CHUNK)], axis=1)
    z128 = jnp.zeros((_N, _D), jnp.float32)
    z1 = jnp.zeros((_N,), jnp.float32)
    w1_0 = jnp.concatenate([W_sl0.T, W_el0.T], axis=1)
    w1_1 = jnp.concatenate([W_sl1.T, W_el1.T], axis=1)

    (deg,) = _sc("deg")(dst, z1)
    deg = deg.reshape(_NW, _N)

    x0 = jnp.concatenate([ent_embed, rel_embed], axis=0)
    si0, t0 = _mm2(x0, w1_0, W_r0.T)
    (agg0,) = _sc("main")(t0, srel, dst, z128)
    agg0 = agg0.reshape(_NC, _N, _D)
    h1 = _apply(agg0, deg, norm, si0)

    x1 = jnp.concatenate([h1, rel_embed], axis=0)
    si1, t1 = _mm2(x1, w1_1, W_r1.T)
    (agg1,) = _sc("main")(t1, srel, dst, z128)
    agg1 = agg1.reshape(_NC, _N, _D)
    h2 = _apply(agg1, deg, norm, si1)
    return h2


# speedup vs baseline: 7.1463x; 1.0161x over previous
import dataclasses
import jax
import jax.numpy as jnp
from jax import lax
from jax.experimental import pallas as pl
from jax.experimental.pallas import tpu as pltpu
from jax.experimental.pallas import tpu_sc as plsc

_N = 10000
_E = 320000
_D = 128
_R = 200
_SLOPE = (1.0 / 8.0 + 1.0 / 3.0) / 2.0
_NC = 2
_NS = 16
_NW = _NC * _NS
_EPT = _E // _NW
_CHUNK = 80
_EPT2 = _EPT
_NCHUNK = _EPT2 // _CHUNK
_NA = _N
_RPT = 624
_CHUNK_DEG = 400
_NT = _N + _R

def _sc_compiler_params():
    cp = pltpu.CompilerParams()
    if "needs_layout_passes" in pltpu.CompilerParams.__dataclass_fields__:
        cp = dataclasses.replace(cp, needs_layout_passes=False)
    return cp


def _mm_block(x_ref, w1_ref, w2_ref, o1_ref, o2_ref):
    x = x_ref[...]
    o1_ref[...] = jnp.dot(x, w1_ref[...], preferred_element_type=jnp.float32)
    o2_ref[...] = jnp.dot(x, w2_ref[...], preferred_element_type=jnp.float32)


def _mm2(x, w1, w2):
    rows = x.shape[0]
    bm = 1024
    grid = pl.cdiv(rows, bm)
    return pl.pallas_call(
        _mm_block,
        grid=(grid,),
        in_specs=[
            pl.BlockSpec((bm, _D), lambda i: (i, 0)),
            pl.BlockSpec((_D, w1.shape[1]), lambda i: (0, 0)),
            pl.BlockSpec((_D, w2.shape[1]), lambda i: (0, 0)),
        ],
        out_specs=[
            pl.BlockSpec((bm, w1.shape[1]), lambda i: (i, 0)),
            pl.BlockSpec((bm, w2.shape[1]), lambda i: (i, 0)),
        ],
        out_shape=[
            jax.ShapeDtypeStruct((rows, w1.shape[1]), jnp.float32),
            jax.ShapeDtypeStruct((rows, w2.shape[1]), jnp.float32),
        ],
    )(x, w1, w2)


def _apply_block(p_ref, deg_ref, norm_ref, si_ref, o_ref):
    agg = p_ref[0] + p_ref[1]
    deg = jnp.sum(deg_ref[...], axis=0)[:, None]
    iso = deg == 0.0
    si = si_ref[...]
    sm = jnp.where(iso, si[:, _D:], si[:, :_D])
    pre = agg * norm_ref[...] + sm
    o_ref[...] = jnp.where(pre >= 0.0, pre, jnp.float32(_SLOPE) * pre)


def _apply(parts, deg, norm, si):
    bm = 1024
    grid = pl.cdiv(_N, bm)
    return pl.pallas_call(
        _apply_block,
        grid=(grid,),
        in_specs=[
            pl.BlockSpec((2, bm, _D), lambda i: (0, i, 0)),
            pl.BlockSpec((_NW, bm), lambda i: (0, i)),
            pl.BlockSpec((bm, 1), lambda i: (i, 0)),
            pl.BlockSpec((bm, 2 * _D), lambda i: (i, 0)),
        ],
        out_specs=pl.BlockSpec((bm, _D), lambda i: (i, 0)),
        out_shape=jax.ShapeDtypeStruct((_N, _D), jnp.float32),
    )(parts, deg, norm, si)


def _make_deg():
    mesh = plsc.VectorSubcoreMesh(core_axis_name="c", subcore_axis_name="s")
    outs = [jax.ShapeDtypeStruct((_NW * _N,), jnp.float32)]
    scratch = [
        pltpu.VMEM((_CHUNK_DEG,), jnp.int32),
        pltpu.VMEM((_N,), jnp.float32),
    ]

    def body(dst_hbm, z1_hbm, deg_hbm, dst_v, deg_t):
        c = lax.axis_index("c")
        s = lax.axis_index("s")
        wid = s * _NC + c
        pltpu.sync_copy(z1_hbm, deg_t)

        @pl.loop(0, _EPT // _CHUNK_DEG)
        def _(i):
            ebase = wid * _EPT + i * _CHUNK_DEG
            pltpu.sync_copy(dst_hbm.at[pl.ds(ebase, _CHUNK_DEG)], dst_v)
            for j in range(_CHUNK_DEG // 16):
                idx16 = dst_v[pl.ds(j * 16, 16)]
                plsc.addupdate_scatter(deg_t, [idx16],
                                       jnp.full((16,), 1.0, jnp.float32))

        pltpu.sync_copy(deg_t, deg_hbm.at[pl.ds(wid * _N, _N)])

    return pl.kernel(body, out_type=outs, mesh=mesh, scratch_types=scratch,
                     compiler_params=_sc_compiler_params())


def _make_sc():
    outs = [jax.ShapeDtypeStruct((_NC * _NA, _D), jnp.float32)]
    scratch = []
    for _slot in range(2):
        scratch += [
            pltpu.VMEM((2, _CHUNK), jnp.int32),
            pltpu.VMEM((_CHUNK,), jnp.int32),
            pltpu.VMEM((_CHUNK, _D), jnp.float32),
            pltpu.VMEM((_CHUNK, _D), jnp.float32),
            pltpu.SemaphoreType.DMA,
            pltpu.SemaphoreType.DMA,
            pltpu.SemaphoreType.DMA,
            pltpu.SemaphoreType.DMA,
        ]
    scratch.append(pltpu.VMEM_SHARED((_NA, _D), jnp.float32))
    mesh = plsc.VectorSubcoreMesh(core_axis_name="c", subcore_axis_name="s")

    def body(t_hbm, srel_hbm, dst_hbm, z128_hbm, agg_hbm,
             isr0, dst0, bufa0, bufb0, sa0, sb0, ss0, si0,
             isr1, dst1, bufa1, bufb1, sa1, sb1, ss1, si1,
             acc_sh):
        c = lax.axis_index("c")
        s = lax.axis_index("s")
        wid = s * _NC + c
        rbase = s * _RPT
        tail = _NA - _NS * _RPT
        slots = ((isr0, dst0, bufa0, bufb0, sa0, sb0, ss0, si0),
                 (isr1, dst1, bufa1, bufb1, sa1, sb1, ss1, si1))

        pltpu.sync_copy(z128_hbm.at[pl.ds(rbase, _RPT)],
                        acc_sh.at[pl.ds(rbase, _RPT)])

        @pl.when(s == _NS - 1)
        def _():
            pltpu.sync_copy(z128_hbm.at[pl.ds(_NS * _RPT, tail)],
                            acc_sh.at[pl.ds(_NS * _RPT, tail)])

        plsc.subcore_barrier()

        def start_isr(k, i):
            isr_v, sem_i = slots[k][0], slots[k][7]
            pltpu.async_copy(srel_hbm.at[wid * _NCHUNK + i], isr_v, sem_i)

        def wait_isr(k, i):
            isr_v, sem_i = slots[k][0], slots[k][7]
            pltpu.make_async_copy(srel_hbm.at[wid * _NCHUNK + i], isr_v,
                                  sem_i).wait()

        def load_dst(k, i):
            dst_v = slots[k][1]
            pltpu.sync_copy(dst_hbm.at[pl.ds(wid * _EPT2 + i * _CHUNK,
                                             _CHUNK)], dst_v)

        def start_gathers(k):
            isr_v, _, buf_a, buf_b, sem_a, sem_b = slots[k][:6]
            pltpu.async_copy(t_hbm.at[isr_v.at[0]], buf_a, sem_a)
            pltpu.async_copy(t_hbm.at[isr_v.at[1]], buf_b, sem_b)

        def wait_gathers(k):
            isr_v, _, buf_a, buf_b, sem_a, sem_b = slots[k][:6]
            pltpu.make_async_copy(t_hbm.at[isr_v.at[0]], buf_a, sem_a).wait()
            pltpu.make_async_copy(t_hbm.at[isr_v.at[1]], buf_b, sem_b).wait()

        def start_scatters(k):
            dst_v, buf_a, buf_b, sem_s = slots[k][1], slots[k][2], slots[k][3], slots[k][6]
            pltpu.async_copy(buf_a, acc_sh.at[dst_v], sem_s, add=True)
            pltpu.async_copy(buf_b, acc_sh.at[dst_v], sem_s, add=True)

        def wait_scatters(k):
            dst_v, buf_a, buf_b, sem_s = slots[k][1], slots[k][2], slots[k][3], slots[k][6]
            pltpu.make_async_copy(buf_a, acc_sh.at[dst_v], sem_s).wait()
            pltpu.make_async_copy(buf_b, acc_sh.at[dst_v], sem_s).wait()

        start_isr(0, 0)
        wait_isr(0, 0)
        load_dst(0, 0)
        start_gathers(0)
        start_isr(1, 1)
        wait_isr(1, 1)
        load_dst(1, 1)
        start_gathers(1)

        @pl.loop(0, _NCHUNK + 1, step=2)
        def _(i):
            wait_gathers(0)
            start_scatters(0)

            @pl.when(i + 2 < _NCHUNK)
            def _():
                start_isr(0, i + 2)

            @pl.when(i + 1 < _NCHUNK)
            def _():
                wait_gathers(1)
                start_scatters(1)

            @pl.when(i + 3 < _NCHUNK)
            def _():
                start_isr(1, i + 3)

            @pl.when(i + 2 < _NCHUNK)
            def _():
                wait_scatters(0)
                wait_isr(0, i + 2)
                load_dst(0, i + 2)
                start_gathers(0)

            @pl.when(i + 3 < _NCHUNK)
            def _():
                wait_scatters(1)
                wait_isr(1, i + 3)
                load_dst(1, i + 3)
                start_gathers(1)

        wait_scatters(1)
        wait_scatters(0)

        plsc.subcore_barrier()
        pltpu.sync_copy(acc_sh.at[pl.ds(rbase, _RPT)],
                        agg_hbm.at[pl.ds(c * _NA + rbase, _RPT)])

        @pl.when(s == _NS - 1)
        def _():
            pltpu.sync_copy(acc_sh.at[pl.ds(_NS * _RPT, tail)],
                            agg_hbm.at[pl.ds(c * _NA + _NS * _RPT, tail)])

    return pl.kernel(body, out_type=outs, mesh=mesh, scratch_types=scratch)


_SC_CACHE = {}


def _sc(kind):
    if kind not in _SC_CACHE:
        _SC_CACHE[kind] = _make_deg() if kind == "deg" else _make_sc()
    return _SC_CACHE[kind]


@jax.jit
def kernel(ent_embed, rel_embed, norm, edge_index, rel_id,
           W_r0, W_sl0, W_el0, W_r1, W_sl1, W_el1):
    pad = _EPT2 - _EPT
    dst = edge_index[1]
    src2 = jnp.concatenate(
        [edge_index[0].reshape(_NW, _EPT),
         jnp.zeros((_NW, pad), jnp.int32)], axis=1)
    rel2 = jnp.concatenate(
        [(rel_id + _N).astype(jnp.int32).reshape(_NW, _EPT),
         jnp.full((_NW, pad), _N, jnp.int32)], axis=1)
    dst2 = jnp.concatenate(
        [dst.reshape(_NW, _EPT),
         jnp.full((_NW, pad), _N, jnp.int32)], axis=1).reshape(-1)
    srel = jnp.stack([src2.reshape(_NW * _NCHUNK, _CHUNK),
                      rel2.reshape(_NW * _NCHUNK, _CHUNK)], axis=1)
    z128 = jnp.zeros((_NA, _D), jnp.float32)
    z1 = jnp.zeros((_N,), jnp.float32)
    w1_0 = jnp.concatenate([W_sl0.T, W_el0.T], axis=1)
    w1_1 = jnp.concatenate([W_sl1.T, W_el1.T], axis=1)

    (deg,) = _sc("deg")(dst, z1)
    deg = deg.reshape(_NW, _N)

    x0 = jnp.concatenate([ent_embed, rel_embed], axis=0)
    si0, t0 = _mm2(x0, w1_0, W_r0.T)
    (agg0,) = _sc("main")(t0, srel, dst2, z128)
    agg0 = agg0.reshape(_NC, _NA, _D)
    h1 = _apply(agg0, deg, norm, si0)

    x1 = jnp.concatenate([h1, rel_embed], axis=0)
    si1, t1 = _mm2(x1, w1_1, W_r1.T)
    (agg1,) = _sc("main")(t1, srel, dst2, z128)
    agg1 = agg1.reshape(_NC, _NA, _D)
    h2 = _apply(agg1, deg, norm, si1)
    return h2


# speedup vs baseline: 8.3335x; 1.1661x over previous
import dataclasses
import jax
import jax.numpy as jnp
from jax import lax
from jax.experimental import pallas as pl
from jax.experimental.pallas import tpu as pltpu
from jax.experimental.pallas import tpu_sc as plsc

_N = 10000
_E = 320000
_D = 128
_R = 200
_SLOPE = (1.0 / 8.0 + 1.0 / 3.0) / 2.0
_NC = 2
_NS = 16
_NW = _NC * _NS
_EPT = _E // _NW
_CHUNK = 80
_EPT2 = _EPT
_NCHUNK = _EPT2 // _CHUNK
_NA = _N
_RPT = 624
_CHUNK_DEG = 400
_NT = _N + _R

def _sc_compiler_params():
    cp = pltpu.CompilerParams()
    if "needs_layout_passes" in pltpu.CompilerParams.__dataclass_fields__:
        cp = dataclasses.replace(cp, needs_layout_passes=False)
    return cp


def _mm_block(x_ref, w1_ref, w2_ref, o1_ref, o2_ref):
    x = x_ref[...]
    o1_ref[...] = jnp.dot(x, w1_ref[...], preferred_element_type=jnp.float32)
    o2_ref[...] = jnp.dot(x, w2_ref[...], preferred_element_type=jnp.float32)


def _mm2(x, w1, w2):
    rows = x.shape[0]
    bm = 1024
    grid = pl.cdiv(rows, bm)
    return pl.pallas_call(
        _mm_block,
        grid=(grid,),
        in_specs=[
            pl.BlockSpec((bm, _D), lambda i: (i, 0)),
            pl.BlockSpec((_D, w1.shape[1]), lambda i: (0, 0)),
            pl.BlockSpec((_D, w2.shape[1]), lambda i: (0, 0)),
        ],
        out_specs=[
            pl.BlockSpec((bm, w1.shape[1]), lambda i: (i, 0)),
            pl.BlockSpec((bm, w2.shape[1]), lambda i: (i, 0)),
        ],
        out_shape=[
            jax.ShapeDtypeStruct((rows, w1.shape[1]), jnp.float32),
            jax.ShapeDtypeStruct((rows, w2.shape[1]), jnp.float32),
        ],
    )(x, w1, w2)


def _apply_block(p_ref, deg_ref, norm_ref, si_ref, o_ref):
    agg = p_ref[0] + p_ref[1]
    deg = jnp.sum(deg_ref[...], axis=0)[:, None]
    iso = deg == 0.0
    si = si_ref[...]
    sm = jnp.where(iso, si[:, _D:], si[:, :_D])
    pre = agg * norm_ref[...] + sm
    o_ref[...] = jnp.where(pre >= 0.0, pre, jnp.float32(_SLOPE) * pre)


def _apply(parts, deg, norm, si):
    bm = 1024
    grid = pl.cdiv(_N, bm)
    return pl.pallas_call(
        _apply_block,
        grid=(grid,),
        in_specs=[
            pl.BlockSpec((2, bm, _D), lambda i: (0, i, 0)),
            pl.BlockSpec((_NW, bm), lambda i: (0, i)),
            pl.BlockSpec((bm, 1), lambda i: (i, 0)),
            pl.BlockSpec((bm, 2 * _D), lambda i: (i, 0)),
        ],
        out_specs=pl.BlockSpec((bm, _D), lambda i: (i, 0)),
        out_shape=jax.ShapeDtypeStruct((_N, _D), jnp.float32),
    )(parts, deg, norm, si)


def _make_deg():
    mesh = plsc.VectorSubcoreMesh(core_axis_name="c", subcore_axis_name="s")
    outs = [jax.ShapeDtypeStruct((_NW * _N,), jnp.float32)]
    scratch = [
        pltpu.VMEM((_CHUNK_DEG,), jnp.int32),
        pltpu.VMEM((_N,), jnp.float32),
    ]

    def body(dst_hbm, z1_hbm, deg_hbm, dst_v, deg_t):
        c = lax.axis_index("c")
        s = lax.axis_index("s")
        wid = s * _NC + c
        pltpu.sync_copy(z1_hbm, deg_t)

        @pl.loop(0, _EPT // _CHUNK_DEG)
        def _(i):
            ebase = wid * _EPT + i * _CHUNK_DEG
            pltpu.sync_copy(dst_hbm.at[pl.ds(ebase, _CHUNK_DEG)], dst_v)
            for j in range(_CHUNK_DEG // 16):
                idx16 = dst_v[pl.ds(j * 16, 16)]
                plsc.addupdate_scatter(deg_t, [idx16],
                                       jnp.full((16,), 1.0, jnp.float32))

        pltpu.sync_copy(deg_t, deg_hbm.at[pl.ds(wid * _N, _N)])

    return pl.kernel(body, out_type=outs, mesh=mesh, scratch_types=scratch,
                     compiler_params=_sc_compiler_params())


def _make_sc():
    outs = [jax.ShapeDtypeStruct((_NC * _NA, _D), jnp.float32)]
    scratch = []
    for _slot in range(2):
        scratch += [
            pltpu.VMEM((2, _CHUNK), jnp.int32),
            pltpu.VMEM((_CHUNK,), jnp.int32),
            pltpu.VMEM((_CHUNK, _D), jnp.float32),
            pltpu.VMEM((_CHUNK, _D), jnp.float32),
            pltpu.SemaphoreType.DMA,
            pltpu.SemaphoreType.DMA,
            pltpu.SemaphoreType.DMA,
            pltpu.SemaphoreType.DMA,
        ]
    scratch.append(pltpu.VMEM_SHARED((_NA, _D), jnp.float32))
    scratch.append(pltpu.VMEM_SHARED((_R, _D), jnp.float32))
    mesh = plsc.VectorSubcoreMesh(core_axis_name="c", subcore_axis_name="s")

    def body(t_hbm, srel_hbm, dst_hbm, z128_hbm, agg_hbm,
             isr0, dst0, bufa0, bufb0, sa0, sb0, ss0, si0,
             isr1, dst1, bufa1, bufb1, sa1, sb1, ss1, si1,
             acc_sh, rw_sh):
        c = lax.axis_index("c")
        s = lax.axis_index("s")
        wid = s * _NC + c
        rbase = s * _RPT
        tail = _NA - _NS * _RPT
        slots = ((isr0, dst0, bufa0, bufb0, sa0, sb0, ss0, si0),
                 (isr1, dst1, bufa1, bufb1, sa1, sb1, ss1, si1))

        pltpu.sync_copy(z128_hbm.at[pl.ds(rbase, _RPT)],
                        acc_sh.at[pl.ds(rbase, _RPT)])

        @pl.when(s == _NS - 1)
        def _():
            pltpu.sync_copy(z128_hbm.at[pl.ds(_NS * _RPT, tail)],
                            acc_sh.at[pl.ds(_NS * _RPT, tail)])

        @pl.when(s == 0)
        def _():
            pltpu.sync_copy(t_hbm.at[pl.ds(_N, _R)], rw_sh)

        plsc.subcore_barrier()

        def start_isr(k, i):
            isr_v, sem_i = slots[k][0], slots[k][7]
            pltpu.async_copy(srel_hbm.at[wid * _NCHUNK + i], isr_v, sem_i)

        def wait_isr(k, i):
            isr_v, sem_i = slots[k][0], slots[k][7]
            pltpu.make_async_copy(srel_hbm.at[wid * _NCHUNK + i], isr_v,
                                  sem_i).wait()

        def load_dst(k, i):
            dst_v = slots[k][1]
            pltpu.sync_copy(dst_hbm.at[pl.ds(wid * _EPT2 + i * _CHUNK,
                                             _CHUNK)], dst_v)

        def start_gathers(k):
            isr_v, _, buf_a, buf_b, sem_a, sem_b = slots[k][:6]
            pltpu.async_copy(t_hbm.at[isr_v.at[0]], buf_a, sem_a)
            pltpu.async_copy(rw_sh.at[isr_v.at[1]], buf_b, sem_b)

        def wait_gathers(k):
            isr_v, _, buf_a, buf_b, sem_a, sem_b = slots[k][:6]
            pltpu.make_async_copy(t_hbm.at[isr_v.at[0]], buf_a, sem_a).wait()
            pltpu.make_async_copy(rw_sh.at[isr_v.at[1]], buf_b, sem_b).wait()

        def start_scatters(k):
            dst_v, buf_a, buf_b, sem_s = slots[k][1], slots[k][2], slots[k][3], slots[k][6]
            pltpu.async_copy(buf_a, acc_sh.at[dst_v], sem_s, add=True)
            pltpu.async_copy(buf_b, acc_sh.at[dst_v], sem_s, add=True)

        def wait_scatters(k):
            dst_v, buf_a, buf_b, sem_s = slots[k][1], slots[k][2], slots[k][3], slots[k][6]
            pltpu.make_async_copy(buf_a, acc_sh.at[dst_v], sem_s).wait()
            pltpu.make_async_copy(buf_b, acc_sh.at[dst_v], sem_s).wait()

        start_isr(0, 0)
        wait_isr(0, 0)
        load_dst(0, 0)
        start_gathers(0)
        start_isr(1, 1)
        wait_isr(1, 1)
        load_dst(1, 1)
        start_gathers(1)

        @pl.loop(0, _NCHUNK + 1, step=2)
        def _(i):
            wait_gathers(0)
            start_scatters(0)

            @pl.when(i + 2 < _NCHUNK)
            def _():
                start_isr(0, i + 2)

            @pl.when(i + 1 < _NCHUNK)
            def _():
                wait_gathers(1)
                start_scatters(1)

            @pl.when(i + 3 < _NCHUNK)
            def _():
                start_isr(1, i + 3)

            @pl.when(i + 2 < _NCHUNK)
            def _():
                wait_scatters(0)
                wait_isr(0, i + 2)
                load_dst(0, i + 2)
                start_gathers(0)

            @pl.when(i + 3 < _NCHUNK)
            def _():
                wait_scatters(1)
                wait_isr(1, i + 3)
                load_dst(1, i + 3)
                start_gathers(1)

        wait_scatters(1)
        wait_scatters(0)

        plsc.subcore_barrier()
        pltpu.sync_copy(acc_sh.at[pl.ds(rbase, _RPT)],
                        agg_hbm.at[pl.ds(c * _NA + rbase, _RPT)])

        @pl.when(s == _NS - 1)
        def _():
            pltpu.sync_copy(acc_sh.at[pl.ds(_NS * _RPT, tail)],
                            agg_hbm.at[pl.ds(c * _NA + _NS * _RPT, tail)])

    return pl.kernel(body, out_type=outs, mesh=mesh, scratch_types=scratch)


_SC_CACHE = {}


def _sc(kind):
    if kind not in _SC_CACHE:
        _SC_CACHE[kind] = _make_deg() if kind == "deg" else _make_sc()
    return _SC_CACHE[kind]


@jax.jit
def kernel(ent_embed, rel_embed, norm, edge_index, rel_id,
           W_r0, W_sl0, W_el0, W_r1, W_sl1, W_el1):
    pad = _EPT2 - _EPT
    dst = edge_index[1]
    src2 = jnp.concatenate(
        [edge_index[0].reshape(_NW, _EPT),
         jnp.zeros((_NW, pad), jnp.int32)], axis=1)
    rel2 = jnp.concatenate(
        [rel_id.astype(jnp.int32).reshape(_NW, _EPT),
         jnp.zeros((_NW, pad), jnp.int32)], axis=1)
    dst2 = jnp.concatenate(
        [dst.reshape(_NW, _EPT),
         jnp.full((_NW, pad), _N, jnp.int32)], axis=1).reshape(-1)
    srel = jnp.stack([src2.reshape(_NW * _NCHUNK, _CHUNK),
                      rel2.reshape(_NW * _NCHUNK, _CHUNK)], axis=1)
    z128 = jnp.zeros((_NA, _D), jnp.float32)
    z1 = jnp.zeros((_N,), jnp.float32)
    w1_0 = jnp.concatenate([W_sl0.T, W_el0.T], axis=1)
    w1_1 = jnp.concatenate([W_sl1.T, W_el1.T], axis=1)

    (deg,) = _sc("deg")(dst, z1)
    deg = deg.reshape(_NW, _N)

    x0 = jnp.concatenate([ent_embed, rel_embed], axis=0)
    si0, t0 = _mm2(x0, w1_0, W_r0.T)
    (agg0,) = _sc("main")(t0, srel, dst2, z128)
    agg0 = agg0.reshape(_NC, _NA, _D)
    h1 = _apply(agg0, deg, norm, si0)

    x1 = jnp.concatenate([h1, rel_embed], axis=0)
    si1, t1 = _mm2(x1, w1_1, W_r1.T)
    (agg1,) = _sc("main")(t1, srel, dst2, z128)
    agg1 = agg1.reshape(_NC, _NA, _D)
    h2 = _apply(agg1, deg, norm, si1)
    return h2
